# Initial kernel scaffold; baseline (speedup 1.0000x reference)
import jax
import jax.numpy as jnp
from jax.experimental import pallas as pl


def kernel(coordinates, plane4_x, plane4_y, plane4_z, B_fourier, iteration, is_training):
    raise NotImplementedError("write your pallas kernel here")


# speedup vs baseline: 30.1725x; 30.1725x over previous
import numpy as np
import jax
import jax.numpy as jnp
from jax.experimental import pallas as pl
from jax.experimental.pallas import tpu as pltpu

CH = 32
G = 32
NT = G * G
A = -0.75


def _cubic(t):
    t2 = t * t
    t3 = t2 * t
    w0 = A * (t3 - 2.0 * t2 + t)
    w1 = (A + 2.0) * t3 - (A + 3.0) * t2 + 1.0
    u = 1.0 - t
    u2 = u * u
    u3 = u2 * u
    w2 = (A + 2.0) * u3 - (A + 3.0) * u2 + 1.0
    w3 = A * (u3 - 2.0 * u2 + u)
    return (w0, w1, w2, w3)


def _axis_wmat(c, B):
    s = (c + 1.0) * (0.5 * (G - 1))
    s0 = jnp.floor(s)
    t = s - s0
    i0 = s0.astype(jnp.int32)
    ws = _cubic(t)
    cols = jax.lax.broadcasted_iota(jnp.int32, (B, G), 1)
    W = jnp.zeros((B, G), jnp.float32)
    for k in range(4):
        ik = jnp.clip(i0 + (k - 1), 0, G - 1)
        W = W + jnp.where(cols == ik[:, None], ws[k][:, None], 0.0)
    return W


def _body(c_ref, tab_ref, bf_ref, o_ref):
    B = o_ref.shape[0]
    x = c_ref[0, :]
    y = c_ref[1, :]
    z = c_ref[2, :]
    Wx = _axis_wmat(x, B)
    Wy = _axis_wmat(y, B)
    Wz = _axis_wmat(z, B)
    Wpx = (Wy[:, :, None] * Wx[:, None, :]).reshape(B, NT)
    Wpy = (Wz[:, :, None] * Wy[:, None, :]).reshape(B, NT)
    Wpz = (Wz[:, :, None] * Wx[:, None, :]).reshape(B, NT)
    W3 = jnp.concatenate([Wpx, Wpy, Wpz], axis=1).astype(jnp.bfloat16)
    emb = jnp.dot(W3, tab_ref[...], preferred_element_type=jnp.float32)
    proj = jnp.dot(emb, bf_ref[...], preferred_element_type=jnp.float32)
    proj = proj * (2.0 * np.pi)
    o_ref[...] = jnp.concatenate([jnp.sin(proj), jnp.cos(proj)], axis=1)


def kernel(coordinates, plane4_x, plane4_y, plane4_z, B_fourier,
           iteration=0, is_training=0):
    N = coordinates.shape[0]
    B = 1024
    ct = coordinates.T
    tabs = jnp.concatenate(
        [jnp.transpose(p, (1, 2, 0)).reshape(NT, CH)
         for p in (plane4_x, plane4_y, plane4_z)], axis=0
    ).astype(jnp.bfloat16)
    grid = N // B
    return pl.pallas_call(
        _body,
        grid=(grid,),
        in_specs=[
            pl.BlockSpec((3, B), lambda i: (0, i)),
            pl.BlockSpec((3 * NT, CH), lambda i: (0, 0)),
            pl.BlockSpec((CH, CH // 2), lambda i: (0, 0)),
        ],
        out_specs=pl.BlockSpec((B, CH), lambda i: (i, 0)),
        out_shape=jax.ShapeDtypeStruct((N, CH), jnp.float32),
    )(ct, tabs, B_fourier)
